# baseline (speedup 1.0000x reference)
import jax
import jax.numpy as jnp
from jax import lax
from jax.experimental import pallas as pl
from jax.experimental.pallas import tpu as pltpu
from jax.experimental.pallas import tpu_sc as plsc

Z = 384
B = 64
K = 8448
N = 16896
EXT_ROWS = 20
NBLK = 26
DBL = 2 * Z
KD = 22 * DBL
ZPAD = NBLK * DBL
CWD = ZPAD + Z
NCHUNK = Z // 16
HALF = NCHUNK // 2

NC = 2
NS = 16
ROWS_PER_W = B // (NC * NS)


def _par2(x):
    return (x.astype(jnp.int32) & 1).astype(jnp.float32)


def _sc_body(bitsd_hbm, bits_hbm, ab_hbm, cb_hbm, p0i_hbm, out_hbm,
             cwd0, cwd1, ab_v, cb_v, p0i_v, m_v, mt_v, ext0, ext1,
             s_tab, s_in0, s_in1, s_out0, s_out1):
    wid = lax.axis_index("s") * NC + lax.axis_index("c")
    cwds = (cwd0, cwd1)
    exts = (ext0, ext1)
    s_ins = (s_in0, s_in1)
    s_outs = (s_out0, s_out1)

    d_tab = [pltpu.async_copy(ab_hbm, ab_v, s_tab),
             pltpu.async_copy(cb_hbm, cb_v, s_tab),
             pltpu.async_copy(p0i_hbm, p0i_v, s_tab)]
    d_in = []
    for k in range(ROWS_PER_W):
        b = wid * ROWS_PER_W + k
        d_in.append(pltpu.async_copy(
            bitsd_hbm.at[pl.ds(b * KD, KD)],
            cwds[k].at[pl.ds(0, KD)], s_ins[k]))

    def zero_tail(j, carry):
        cwd0[pl.ds(ZPAD + j * 16, 16)] = jnp.zeros((16,), jnp.float32)
        cwd1[pl.ds(ZPAD + j * 16, 16)] = jnp.zeros((16,), jnp.float32)
        return carry
    lax.fori_loop(0, NCHUNK, zero_tail, 0)
    for d in d_tab:
        d.wait()

    zero16 = jnp.zeros((16,), jnp.float32)
    d_out = []
    for k in range(ROWS_PER_W):
        b = wid * ROWS_PER_W + k
        cwd_v = cwds[k]
        ext_v = exts[k]
        d_in[k].wait()

        for r in range(4):
            for h in range(2):
                def entry1(e, accs, r=r, h=h, cwd_v=cwd_v):
                    base = ab_v[pl.ds((r * 22 + e) * 16, 16)]
                    return tuple(
                        a + plsc.load_gather(cwd_v, [base + (h * HALF + j) * 16])
                        for j, a in enumerate(accs))
                accs = lax.fori_loop(0, 22, entry1, (zero16,) * HALF)
                for j, a in enumerate(accs):
                    m_v[pl.ds(r * Z + (h * HALF + j) * 16, 16)] = _par2(a)

        def stage2(j, carry):
            off = j * 16
            m0 = m_v[pl.ds(0 * Z + off, 16)]
            m1 = m_v[pl.ds(1 * Z + off, 16)]
            m2 = m_v[pl.ds(2 * Z + off, 16)]
            m3 = m_v[pl.ds(3 * Z + off, 16)]
            mt_v[pl.ds(off, 16)] = _par2(m0 + m1 + m2 + m3)
            return carry
        lax.fori_loop(0, NCHUNK, stage2, 0)

        def stage2b(j, carry, cwd_v=cwd_v):
            off = j * 16
            pidx = p0i_v[pl.ds(off, 16)]
            p0 = plsc.load_gather(mt_v, [pidx])
            m1 = m_v[pl.ds(1 * Z + off, 16)]
            m2 = m_v[pl.ds(2 * Z + off, 16)]
            m3 = m_v[pl.ds(3 * Z + off, 16)]
            p1 = _par2(m1 + m2 + m3)
            p3 = _par2(m3 + p0)
            p2 = _par2(m2 + p3)
            for i, p in enumerate((p0, p1, p2, p3)):
                cwd_v[pl.ds((22 + i) * DBL + off, 16)] = p
                cwd_v[pl.ds((22 + i) * DBL + Z + off, 16)] = p
            return carry
        lax.fori_loop(0, NCHUNK, stage2b, 0)

        def ext_row(r, carry, cwd_v=cwd_v, ext_v=ext_v):
            for h in range(2):
                def entry3(e, accs, h=h):
                    base = cb_v[pl.ds((r * 4 + e) * 16, 16)]
                    return tuple(
                        a + plsc.load_gather(cwd_v, [base + (h * HALF + j) * 16])
                        for j, a in enumerate(accs))
                accs = lax.fori_loop(0, 4, entry3, (zero16,) * HALF)
                for j, a in enumerate(accs):
                    ext_v[pl.ds(r * Z + (h * HALF + j) * 16, 16)] = _par2(a)
            return carry
        lax.fori_loop(0, EXT_ROWS, ext_row, 0)

        ob = b * N
        d_out.append(pltpu.async_copy(
            bits_hbm.at[pl.ds(b * K + 2 * Z, K - 2 * Z)],
            out_hbm.at[pl.ds(ob, K - 2 * Z)], s_outs[k]))
        for i in range(4):
            d_out.append(pltpu.async_copy(
                cwd_v.at[pl.ds((22 + i) * DBL, Z)],
                out_hbm.at[pl.ds(ob + K - 2 * Z + i * Z, Z)], s_outs[k]))
        d_out.append(pltpu.async_copy(
            ext_v, out_hbm.at[pl.ds(ob + K + 2 * Z, EXT_ROWS * Z)], s_outs[k]))
    for d in d_out:
        d.wait()


def kernel(inputs, A_r, A_c, A_s, C_r, C_c, C_s):
    bits = inputs.astype(jnp.float32)
    bitsd = jnp.concatenate(
        [bits.reshape(B, 22, Z)] * 2, axis=-1).reshape(B * KD)
    bits = bits.reshape(B * K)
    ar = jnp.asarray(A_r, jnp.int32)
    ac = jnp.asarray(A_c, jnp.int32)
    ash = jnp.asarray(A_s, jnp.int32)
    cc = jnp.asarray(C_c, jnp.int32)
    cs = jnp.asarray(C_s, jnp.int32)
    del C_r
    na = ar.shape[0]

    iota16 = jnp.arange(16, dtype=jnp.int32)
    perm = jnp.argsort(ar, stable=True)
    r_sorted = ar[perm]
    first = jnp.searchsorted(r_sorted, jnp.arange(4, dtype=jnp.int32))
    rank = jnp.arange(na, dtype=jnp.int32) - first[r_sorted]
    slots = r_sorted * 22 + rank
    a_base = (ac * DBL + ash)[perm][:, None] + iota16[None, :]
    ab = jnp.full((4 * 22, 16), ZPAD, jnp.int32).at[slots].set(a_base)
    ab = ab.reshape(-1)
    cb = ((cc[:4 * EXT_ROWS] * DBL + cs[:4 * EXT_ROWS])[:, None]
          + iota16[None, :]).reshape(-1)
    iota = jnp.arange(Z, dtype=jnp.int32)
    p0i = (iota + Z - 1) % Z

    mesh = plsc.VectorSubcoreMesh(core_axis_name="c", subcore_axis_name="s")
    out = pl.kernel(
        _sc_body,
        out_type=jax.ShapeDtypeStruct((B * N,), jnp.float32),
        mesh=mesh,
        compiler_params=pltpu.CompilerParams(needs_layout_passes=False),
        scratch_types=[
            pltpu.VMEM((CWD,), jnp.float32),
            pltpu.VMEM((CWD,), jnp.float32),
            pltpu.VMEM((4 * 22 * 16,), jnp.int32),
            pltpu.VMEM((4 * EXT_ROWS * 16,), jnp.int32),
            pltpu.VMEM((Z,), jnp.int32),
            pltpu.VMEM((4 * Z,), jnp.float32),
            pltpu.VMEM((Z,), jnp.float32),
            pltpu.VMEM((EXT_ROWS * Z,), jnp.float32),
            pltpu.VMEM((EXT_ROWS * Z,), jnp.float32),
            pltpu.SemaphoreType.DMA,
            pltpu.SemaphoreType.DMA,
            pltpu.SemaphoreType.DMA,
            pltpu.SemaphoreType.DMA,
            pltpu.SemaphoreType.DMA,
        ],
    )(bitsd, bits, ab, cb, p0i)
    return out.reshape(B, N)


# speedup vs baseline: 1.9097x; 1.9097x over previous
import jax
import jax.numpy as jnp
from jax import lax
from jax.experimental import pallas as pl
from jax.experimental.pallas import tpu as pltpu
from jax.experimental.pallas import tpu_sc as plsc

Z = 384
B = 64
K = 8448
N = 16896
EXT_ROWS = 20
NBLK = 26
DBL = 2 * Z
KD = 22 * DBL
ZPAD = NBLK * DBL
CWD = ZPAD + Z
NCHUNK = Z // 16
HALF = NCHUNK // 2

NC = 2
NS = 16
ROWS_PER_W = B // (NC * NS)


def _par2(x):
    return (x.astype(jnp.int32) & 1).astype(jnp.float32)


def _sc_body(bitsd_hbm, bits_hbm, ab_hbm, cb_hbm, p0i_hbm, out_hbm,
             cwd0, cwd1, sys0, sys1, ab_v, cb_v, p0i_v, m_v, mt_v, ext0, ext1,
             s_tab, s_in0, s_in1, s_out0, s_out1):
    wid = lax.axis_index("s") * NC + lax.axis_index("c")
    cwds = (cwd0, cwd1)
    syss = (sys0, sys1)
    exts = (ext0, ext1)
    s_ins = (s_in0, s_in1)
    s_outs = (s_out0, s_out1)

    d_tab = [pltpu.async_copy(ab_hbm, ab_v, s_tab),
             pltpu.async_copy(cb_hbm, cb_v, s_tab),
             pltpu.async_copy(p0i_hbm, p0i_v, s_tab)]
    d_in = []
    for k in range(ROWS_PER_W):
        b = wid * ROWS_PER_W + k
        d_in.append(pltpu.async_copy(
            bitsd_hbm.at[pl.ds(b * KD, KD)],
            cwds[k].at[pl.ds(0, KD)], s_ins[k]))
        d_in.append(pltpu.async_copy(
            bits_hbm.at[pl.ds(b * K + 2 * Z, K - 2 * Z)],
            syss[k], s_ins[k]))

    def zero_tail(j, carry):
        cwd0[pl.ds(ZPAD + j * 16, 16)] = jnp.zeros((16,), jnp.float32)
        cwd1[pl.ds(ZPAD + j * 16, 16)] = jnp.zeros((16,), jnp.float32)
        return carry
    lax.fori_loop(0, NCHUNK, zero_tail, 0)
    for d in d_tab:
        d.wait()

    zero16 = jnp.zeros((16,), jnp.float32)
    d_out = []
    for k in range(ROWS_PER_W):
        b = wid * ROWS_PER_W + k
        cwd_v = cwds[k]
        ext_v = exts[k]
        d_in[2 * k].wait()
        d_in[2 * k + 1].wait()

        for r in range(4):
            for h in range(2):
                def entry1(e, accs, r=r, h=h, cwd_v=cwd_v):
                    base = ab_v[pl.ds((r * 22 + e) * 16, 16)]
                    return tuple(
                        a + plsc.load_gather(cwd_v, [base + (h * HALF + j) * 16])
                        for j, a in enumerate(accs))
                accs = lax.fori_loop(0, 22, entry1, (zero16,) * HALF)
                for j, a in enumerate(accs):
                    m_v[pl.ds(r * Z + (h * HALF + j) * 16, 16)] = _par2(a)

        def stage2(j, carry):
            off = j * 16
            m0 = m_v[pl.ds(0 * Z + off, 16)]
            m1 = m_v[pl.ds(1 * Z + off, 16)]
            m2 = m_v[pl.ds(2 * Z + off, 16)]
            m3 = m_v[pl.ds(3 * Z + off, 16)]
            mt_v[pl.ds(off, 16)] = _par2(m0 + m1 + m2 + m3)
            return carry
        lax.fori_loop(0, NCHUNK, stage2, 0)

        def stage2b(j, carry, cwd_v=cwd_v):
            off = j * 16
            pidx = p0i_v[pl.ds(off, 16)]
            p0 = plsc.load_gather(mt_v, [pidx])
            m1 = m_v[pl.ds(1 * Z + off, 16)]
            m2 = m_v[pl.ds(2 * Z + off, 16)]
            m3 = m_v[pl.ds(3 * Z + off, 16)]
            p1 = _par2(m1 + m2 + m3)
            p3 = _par2(m3 + p0)
            p2 = _par2(m2 + p3)
            for i, p in enumerate((p0, p1, p2, p3)):
                cwd_v[pl.ds((22 + i) * DBL + off, 16)] = p
                cwd_v[pl.ds((22 + i) * DBL + Z + off, 16)] = p
            return carry
        lax.fori_loop(0, NCHUNK, stage2b, 0)

        def ext_row(r, carry, cwd_v=cwd_v, ext_v=ext_v):
            for h in range(2):
                def entry3(e, accs, h=h):
                    base = cb_v[pl.ds((r * 4 + e) * 16, 16)]
                    return tuple(
                        a + plsc.load_gather(cwd_v, [base + (h * HALF + j) * 16])
                        for j, a in enumerate(accs))
                accs = lax.fori_loop(0, 4, entry3, (zero16,) * HALF)
                for j, a in enumerate(accs):
                    ext_v[pl.ds(r * Z + (h * HALF + j) * 16, 16)] = _par2(a)
            return carry
        lax.fori_loop(0, EXT_ROWS, ext_row, 0)

        ob = b * N
        d_out.append(pltpu.async_copy(
            syss[k], out_hbm.at[pl.ds(ob, K - 2 * Z)], s_outs[k]))
        for i in range(4):
            d_out.append(pltpu.async_copy(
                cwd_v.at[pl.ds((22 + i) * DBL, Z)],
                out_hbm.at[pl.ds(ob + K - 2 * Z + i * Z, Z)], s_outs[k]))
        d_out.append(pltpu.async_copy(
            ext_v, out_hbm.at[pl.ds(ob + K + 2 * Z, EXT_ROWS * Z)], s_outs[k]))
    for d in d_out:
        d.wait()


def kernel(inputs, A_r, A_c, A_s, C_r, C_c, C_s):
    bits = inputs.astype(jnp.float32)
    bitsd = jnp.concatenate(
        [bits.reshape(B, 22, Z)] * 2, axis=-1).reshape(B * KD)
    bits = bits.reshape(B * K)
    ar = jnp.asarray(A_r, jnp.int32)
    ac = jnp.asarray(A_c, jnp.int32)
    ash = jnp.asarray(A_s, jnp.int32)
    cc = jnp.asarray(C_c, jnp.int32)
    cs = jnp.asarray(C_s, jnp.int32)
    del C_r
    na = ar.shape[0]

    iota16 = jnp.arange(16, dtype=jnp.int32)
    perm = jnp.argsort(ar, stable=True)
    r_sorted = ar[perm]
    first = jnp.searchsorted(r_sorted, jnp.arange(4, dtype=jnp.int32))
    rank = jnp.arange(na, dtype=jnp.int32) - first[r_sorted]
    slots = r_sorted * 22 + rank
    a_base = (ac * DBL + ash)[perm][:, None] + iota16[None, :]
    ab = jnp.full((4 * 22, 16), ZPAD, jnp.int32).at[slots].set(a_base)
    ab = ab.reshape(-1)
    cb = ((cc[:4 * EXT_ROWS] * DBL + cs[:4 * EXT_ROWS])[:, None]
          + iota16[None, :]).reshape(-1)
    iota = jnp.arange(Z, dtype=jnp.int32)
    p0i = (iota + Z - 1) % Z

    mesh = plsc.VectorSubcoreMesh(core_axis_name="c", subcore_axis_name="s")
    out = pl.kernel(
        _sc_body,
        out_type=jax.ShapeDtypeStruct((B * N,), jnp.float32),
        mesh=mesh,
        compiler_params=pltpu.CompilerParams(needs_layout_passes=False),
        scratch_types=[
            pltpu.VMEM((CWD,), jnp.float32),
            pltpu.VMEM((CWD,), jnp.float32),
            pltpu.VMEM((K - 2 * Z,), jnp.float32),
            pltpu.VMEM((K - 2 * Z,), jnp.float32),
            pltpu.VMEM((4 * 22 * 16,), jnp.int32),
            pltpu.VMEM((4 * EXT_ROWS * 16,), jnp.int32),
            pltpu.VMEM((Z,), jnp.int32),
            pltpu.VMEM((4 * Z,), jnp.float32),
            pltpu.VMEM((Z,), jnp.float32),
            pltpu.VMEM((EXT_ROWS * Z,), jnp.float32),
            pltpu.VMEM((EXT_ROWS * Z,), jnp.float32),
            pltpu.SemaphoreType.DMA,
            pltpu.SemaphoreType.DMA,
            pltpu.SemaphoreType.DMA,
            pltpu.SemaphoreType.DMA,
            pltpu.SemaphoreType.DMA,
        ],
    )(bitsd, bits, ab, cb, p0i)
    return out.reshape(B, N)


# speedup vs baseline: 2.3287x; 1.2194x over previous
import jax
import jax.numpy as jnp
from jax import lax
from jax.experimental import pallas as pl
from jax.experimental.pallas import tpu as pltpu
from jax.experimental.pallas import tpu_sc as plsc

Z = 384
B = 64
K = 8448
N = 16896
EXT_ROWS = 20
NBLK = 26
DBL = 2 * Z
KD = 22 * DBL
ZPAD = NBLK * DBL
CWD = ZPAD + Z
NCHUNK = Z // 16
HALF = NCHUNK // 2

NC = 2
NS = 16
B_SC = 32
B_TC = B - B_SC
ROWS_PER_W = B_SC // (NC * NS)


def _par2(x):
    return (x.astype(jnp.int32) & 1).astype(jnp.float32)


def _sc_body(bitsd_hbm, bits_hbm, ab_hbm, cb_hbm, p0i_hbm, out_hbm,
             cwd0, cwd1, sys0, sys1, ab_v, cb_v, p0i_v, m_v, mt_v, ext0, ext1,
             s_tab, s_in0, s_in1, s_out0, s_out1):
    wid = lax.axis_index("s") * NC + lax.axis_index("c")
    cwds = (cwd0, cwd1)
    syss = (sys0, sys1)
    exts = (ext0, ext1)
    s_ins = (s_in0, s_in1)
    s_outs = (s_out0, s_out1)

    d_tab = [pltpu.async_copy(ab_hbm, ab_v, s_tab),
             pltpu.async_copy(cb_hbm, cb_v, s_tab),
             pltpu.async_copy(p0i_hbm, p0i_v, s_tab)]
    d_in = []
    for k in range(ROWS_PER_W):
        b = wid * ROWS_PER_W + k
        d_in.append(pltpu.async_copy(
            bitsd_hbm.at[pl.ds(b * KD, KD)],
            cwds[k].at[pl.ds(0, KD)], s_ins[k]))
        d_in.append(pltpu.async_copy(
            bits_hbm.at[pl.ds(b * K + 2 * Z, K - 2 * Z)],
            syss[k], s_ins[k]))

    def zero_tail(j, carry):
        cwd0[pl.ds(ZPAD + j * 16, 16)] = jnp.zeros((16,), jnp.float32)
        cwd1[pl.ds(ZPAD + j * 16, 16)] = jnp.zeros((16,), jnp.float32)
        return carry
    lax.fori_loop(0, NCHUNK, zero_tail, 0)
    for d in d_tab:
        d.wait()

    zero16 = jnp.zeros((16,), jnp.float32)
    d_out = []
    for k in range(ROWS_PER_W):
        b = wid * ROWS_PER_W + k
        cwd_v = cwds[k]
        ext_v = exts[k]
        d_in[2 * k].wait()
        d_in[2 * k + 1].wait()

        for r in range(4):
            for h in range(2):
                def entry1(e, accs, r=r, h=h, cwd_v=cwd_v):
                    base = ab_v[pl.ds((r * 22 + e) * 16, 16)]
                    return tuple(
                        a + plsc.load_gather(cwd_v, [base + (h * HALF + j) * 16])
                        for j, a in enumerate(accs))
                accs = lax.fori_loop(0, 22, entry1, (zero16,) * HALF)
                for j, a in enumerate(accs):
                    m_v[pl.ds(r * Z + (h * HALF + j) * 16, 16)] = _par2(a)

        def stage2(j, carry):
            off = j * 16
            m0 = m_v[pl.ds(0 * Z + off, 16)]
            m1 = m_v[pl.ds(1 * Z + off, 16)]
            m2 = m_v[pl.ds(2 * Z + off, 16)]
            m3 = m_v[pl.ds(3 * Z + off, 16)]
            mt_v[pl.ds(off, 16)] = _par2(m0 + m1 + m2 + m3)
            return carry
        lax.fori_loop(0, NCHUNK, stage2, 0)

        def stage2b(j, carry, cwd_v=cwd_v):
            off = j * 16
            pidx = p0i_v[pl.ds(off, 16)]
            p0 = plsc.load_gather(mt_v, [pidx])
            m1 = m_v[pl.ds(1 * Z + off, 16)]
            m2 = m_v[pl.ds(2 * Z + off, 16)]
            m3 = m_v[pl.ds(3 * Z + off, 16)]
            p1 = _par2(m1 + m2 + m3)
            p3 = _par2(m3 + p0)
            p2 = _par2(m2 + p3)
            for i, p in enumerate((p0, p1, p2, p3)):
                cwd_v[pl.ds((22 + i) * DBL + off, 16)] = p
                cwd_v[pl.ds((22 + i) * DBL + Z + off, 16)] = p
            return carry
        lax.fori_loop(0, NCHUNK, stage2b, 0)

        def ext_row(r, carry, cwd_v=cwd_v, ext_v=ext_v):
            for h in range(2):
                def entry3(e, accs, h=h):
                    base = cb_v[pl.ds((r * 4 + e) * 16, 16)]
                    return tuple(
                        a + plsc.load_gather(cwd_v, [base + (h * HALF + j) * 16])
                        for j, a in enumerate(accs))
                accs = lax.fori_loop(0, 4, entry3, (zero16,) * HALF)
                for j, a in enumerate(accs):
                    ext_v[pl.ds(r * Z + (h * HALF + j) * 16, 16)] = _par2(a)
            return carry
        lax.fori_loop(0, EXT_ROWS, ext_row, 0)

        ob = b * N
        d_out.append(pltpu.async_copy(
            syss[k], out_hbm.at[pl.ds(ob, K - 2 * Z)], s_outs[k]))
        for i in range(4):
            d_out.append(pltpu.async_copy(
                cwd_v.at[pl.ds((22 + i) * DBL, Z)],
                out_hbm.at[pl.ds(ob + K - 2 * Z + i * Z, Z)], s_outs[k]))
        d_out.append(pltpu.async_copy(
            ext_v, out_hbm.at[pl.ds(ob + K + 2 * Z, EXT_ROWS * Z)], s_outs[k]))
    for d in d_out:
        d.wait()


def _tc_body(nb, na, bits_ref, ar_ref, ac_ref, as_ref, cc_ref, cs_ref,
             out_ref, macc_ref, cw_ref):
    cw_ref[:, :K] = bits_ref[...]
    macc_ref[...] = jnp.zeros((nb, 4 * Z), jnp.float32)

    def body1(i, carry):
        c = ac_ref[i]
        s = as_ref[i]
        r = ar_ref[i]
        blk = bits_ref[:, pl.ds(pl.multiple_of(c * Z, 128), Z)]
        rolled = pltpu.roll(blk, (Z - s) % Z, axis=1)
        off = pl.multiple_of(r * Z, 128)
        macc_ref[:, pl.ds(off, Z)] = macc_ref[:, pl.ds(off, Z)] + rolled
        return carry

    lax.fori_loop(0, na, body1, 0)

    m = jnp.mod(macc_ref[...], 2.0)
    m0 = m[:, 0 * Z:1 * Z]
    m1 = m[:, 1 * Z:2 * Z]
    m2 = m[:, 2 * Z:3 * Z]
    m3 = m[:, 3 * Z:4 * Z]
    mtot = jnp.mod(m0 + m1 + m2 + m3, 2.0)
    p0 = pltpu.roll(mtot, 1, axis=1)
    p1 = jnp.mod(m1 + m2 + m3, 2.0)
    p3 = jnp.mod(m3 + p0, 2.0)
    p2 = jnp.mod(m2 + p3, 2.0)
    cw_ref[:, K + 0 * Z:K + 1 * Z] = p0
    cw_ref[:, K + 1 * Z:K + 2 * Z] = p1
    cw_ref[:, K + 2 * Z:K + 3 * Z] = p2
    cw_ref[:, K + 3 * Z:K + 4 * Z] = p3

    out_ref[:, :K - 2 * Z] = bits_ref[:, 2 * Z:]
    out_ref[:, K - 2 * Z:K + 2 * Z] = cw_ref[:, K:K + 4 * Z]

    for r in range(EXT_ROWS):
        acc = jnp.zeros((nb, Z), jnp.float32)
        for e in range(4):
            i = 4 * r + e
            c = cc_ref[i]
            s = cs_ref[i]
            blk = cw_ref[:, pl.ds(pl.multiple_of(c * Z, 128), Z)]
            acc = acc + pltpu.roll(blk, (Z - s) % Z, axis=1)
        out_ref[:, K + 2 * Z + r * Z:K + 2 * Z + (r + 1) * Z] = jnp.mod(acc, 2.0)


def kernel(inputs, A_r, A_c, A_s, C_r, C_c, C_s):
    bits_all = inputs.astype(jnp.float32)
    bits2d = bits_all[:B_SC]
    bitsd = jnp.concatenate(
        [bits2d.reshape(B_SC, 22, Z)] * 2, axis=-1).reshape(B_SC * KD)
    bits = bits2d.reshape(B_SC * K)
    ar = jnp.asarray(A_r, jnp.int32)
    ac = jnp.asarray(A_c, jnp.int32)
    ash = jnp.asarray(A_s, jnp.int32)
    cc = jnp.asarray(C_c, jnp.int32)
    cs = jnp.asarray(C_s, jnp.int32)
    del C_r
    na = ar.shape[0]

    iota16 = jnp.arange(16, dtype=jnp.int32)
    perm = jnp.argsort(ar, stable=True)
    r_sorted = ar[perm]
    first = jnp.searchsorted(r_sorted, jnp.arange(4, dtype=jnp.int32))
    rank = jnp.arange(na, dtype=jnp.int32) - first[r_sorted]
    slots = r_sorted * 22 + rank
    a_base = (ac * DBL + ash)[perm][:, None] + iota16[None, :]
    ab = jnp.full((4 * 22, 16), ZPAD, jnp.int32).at[slots].set(a_base)
    ab = ab.reshape(-1)
    cb = ((cc[:4 * EXT_ROWS] * DBL + cs[:4 * EXT_ROWS])[:, None]
          + iota16[None, :]).reshape(-1)
    iota = jnp.arange(Z, dtype=jnp.int32)
    p0i = (iota + Z - 1) % Z

    mesh = plsc.VectorSubcoreMesh(core_axis_name="c", subcore_axis_name="s")
    out_sc = pl.kernel(
        _sc_body,
        out_type=jax.ShapeDtypeStruct((B_SC * N,), jnp.float32),
        mesh=mesh,
        compiler_params=pltpu.CompilerParams(needs_layout_passes=False),
        scratch_types=[
            pltpu.VMEM((CWD,), jnp.float32),
            pltpu.VMEM((CWD,), jnp.float32),
            pltpu.VMEM((K - 2 * Z,), jnp.float32),
            pltpu.VMEM((K - 2 * Z,), jnp.float32),
            pltpu.VMEM((4 * 22 * 16,), jnp.int32),
            pltpu.VMEM((4 * EXT_ROWS * 16,), jnp.int32),
            pltpu.VMEM((Z,), jnp.int32),
            pltpu.VMEM((4 * Z,), jnp.float32),
            pltpu.VMEM((Z,), jnp.float32),
            pltpu.VMEM((EXT_ROWS * Z,), jnp.float32),
            pltpu.VMEM((EXT_ROWS * Z,), jnp.float32),
            pltpu.SemaphoreType.DMA,
            pltpu.SemaphoreType.DMA,
            pltpu.SemaphoreType.DMA,
            pltpu.SemaphoreType.DMA,
            pltpu.SemaphoreType.DMA,
        ],
    )(bitsd, bits, ab, cb, p0i)

    import functools
    body = functools.partial(_tc_body, B_TC, na)
    smem = pl.BlockSpec(memory_space=pltpu.SMEM)
    out_tc = pl.pallas_call(
        body,
        out_shape=jax.ShapeDtypeStruct((B_TC, N), jnp.float32),
        in_specs=[pl.BlockSpec(memory_space=pltpu.VMEM),
                  smem, smem, smem, smem, smem],
        out_specs=pl.BlockSpec(memory_space=pltpu.VMEM),
        scratch_shapes=[pltpu.VMEM((B_TC, 4 * Z), jnp.float32),
                        pltpu.VMEM((B_TC, NBLK * Z), jnp.float32)],
    )(bits_all[B_SC:], ar, ac, ash, cc, cs)

    return jnp.concatenate([out_sc.reshape(B_SC, N), out_tc], axis=0)


# speedup vs baseline: 2.3466x; 1.0077x over previous
import jax
import jax.numpy as jnp
from jax import lax
from jax.experimental import pallas as pl
from jax.experimental.pallas import tpu as pltpu
from jax.experimental.pallas import tpu_sc as plsc

Z = 384
B = 64
K = 8448
N = 16896
EXT_ROWS = 20
NBLK = 26
DBL = 2 * Z
KD = 22 * DBL
ZPAD = NBLK * DBL
CWD = ZPAD + Z
NCHUNK = Z // 16
HALF = NCHUNK // 2

NC = 2
NS = 16
B_SC = 32
B_TC = B - B_SC
ROWS_PER_W = B_SC // (NC * NS)


def _par2(x):
    return (x.astype(jnp.int32) & 1).astype(jnp.float32)


def _sc_body(bits_hbm, ab_hbm, cb_hbm, p0i_hbm, out_hbm,
             cwd0, cwd1, bv0, bv1, ab_v, cb_v, p0i_v, m_v, mt_v, ext0, ext1,
             s_tab, s_in0, s_in1, s_out0, s_out1):
    wid = lax.axis_index("s") * NC + lax.axis_index("c")
    cwds = (cwd0, cwd1)
    bvs = (bv0, bv1)
    exts = (ext0, ext1)
    s_ins = (s_in0, s_in1)
    s_outs = (s_out0, s_out1)

    d_tab = [pltpu.async_copy(ab_hbm, ab_v, s_tab),
             pltpu.async_copy(cb_hbm, cb_v, s_tab),
             pltpu.async_copy(p0i_hbm, p0i_v, s_tab)]
    d_in = []
    for k in range(ROWS_PER_W):
        b = wid * ROWS_PER_W + k
        d_in.append(pltpu.async_copy(
            bits_hbm.at[pl.ds(b * K, K)], bvs[k], s_ins[k]))

    def zero_tail(j, carry):
        cwd0[pl.ds(ZPAD + j * 16, 16)] = jnp.zeros((16,), jnp.float32)
        cwd1[pl.ds(ZPAD + j * 16, 16)] = jnp.zeros((16,), jnp.float32)
        return carry
    lax.fori_loop(0, NCHUNK, zero_tail, 0)
    for d in d_tab:
        d.wait()

    zero16 = jnp.zeros((16,), jnp.float32)
    d_out = []
    for k in range(ROWS_PER_W):
        b = wid * ROWS_PER_W + k
        cwd_v = cwds[k]
        bits_v = bvs[k]
        ext_v = exts[k]
        d_in[k].wait()

        def dup(c, carry, cwd_v=cwd_v, bits_v=bits_v):
            vs = [bits_v[pl.ds(c * Z + j * 16, 16)] for j in range(NCHUNK)]
            for j in range(NCHUNK):
                cwd_v[pl.ds(c * DBL + j * 16, 16)] = vs[j]
                cwd_v[pl.ds(c * DBL + Z + j * 16, 16)] = vs[j]
            return carry
        lax.fori_loop(0, 22, dup, 0)

        for r in range(4):
            for h in range(2):
                def entry1(e, accs, r=r, h=h, cwd_v=cwd_v):
                    base = ab_v[pl.ds((r * 22 + e) * 16, 16)]
                    return tuple(
                        a + plsc.load_gather(cwd_v, [base + (h * HALF + j) * 16])
                        for j, a in enumerate(accs))
                accs = lax.fori_loop(0, 22, entry1, (zero16,) * HALF)
                for j, a in enumerate(accs):
                    m_v[pl.ds(r * Z + (h * HALF + j) * 16, 16)] = _par2(a)

        def stage2(j, carry):
            off = j * 16
            m0 = m_v[pl.ds(0 * Z + off, 16)]
            m1 = m_v[pl.ds(1 * Z + off, 16)]
            m2 = m_v[pl.ds(2 * Z + off, 16)]
            m3 = m_v[pl.ds(3 * Z + off, 16)]
            mt_v[pl.ds(off, 16)] = _par2(m0 + m1 + m2 + m3)
            return carry
        lax.fori_loop(0, NCHUNK, stage2, 0)

        def stage2b(j, carry, cwd_v=cwd_v):
            off = j * 16
            pidx = p0i_v[pl.ds(off, 16)]
            p0 = plsc.load_gather(mt_v, [pidx])
            m1 = m_v[pl.ds(1 * Z + off, 16)]
            m2 = m_v[pl.ds(2 * Z + off, 16)]
            m3 = m_v[pl.ds(3 * Z + off, 16)]
            p1 = _par2(m1 + m2 + m3)
            p3 = _par2(m3 + p0)
            p2 = _par2(m2 + p3)
            for i, p in enumerate((p0, p1, p2, p3)):
                cwd_v[pl.ds((22 + i) * DBL + off, 16)] = p
                cwd_v[pl.ds((22 + i) * DBL + Z + off, 16)] = p
            return carry
        lax.fori_loop(0, NCHUNK, stage2b, 0)

        def ext_row(r, carry, cwd_v=cwd_v, ext_v=ext_v):
            for h in range(2):
                def entry3(e, accs, h=h):
                    base = cb_v[pl.ds((r * 4 + e) * 16, 16)]
                    return tuple(
                        a + plsc.load_gather(cwd_v, [base + (h * HALF + j) * 16])
                        for j, a in enumerate(accs))
                accs = lax.fori_loop(0, 4, entry3, (zero16,) * HALF)
                for j, a in enumerate(accs):
                    ext_v[pl.ds(r * Z + (h * HALF + j) * 16, 16)] = _par2(a)
            return carry
        lax.fori_loop(0, EXT_ROWS, ext_row, 0)

        ob = b * N
        d_out.append(pltpu.async_copy(
            bits_v.at[pl.ds(2 * Z, K - 2 * Z)],
            out_hbm.at[pl.ds(ob, K - 2 * Z)], s_outs[k]))
        for i in range(4):
            d_out.append(pltpu.async_copy(
                cwd_v.at[pl.ds((22 + i) * DBL, Z)],
                out_hbm.at[pl.ds(ob + K - 2 * Z + i * Z, Z)], s_outs[k]))
        d_out.append(pltpu.async_copy(
            ext_v, out_hbm.at[pl.ds(ob + K + 2 * Z, EXT_ROWS * Z)], s_outs[k]))
    for d in d_out:
        d.wait()


def _tc_body(nb, na, bits_ref, ar_ref, ac_ref, as_ref, cc_ref, cs_ref,
             out_ref, macc_ref, cw_ref):
    cw_ref[:, :K] = bits_ref[...]
    macc_ref[...] = jnp.zeros((nb, 4 * Z), jnp.float32)

    def body1(i, carry):
        c = ac_ref[i]
        s = as_ref[i]
        r = ar_ref[i]
        blk = bits_ref[:, pl.ds(pl.multiple_of(c * Z, 128), Z)]
        rolled = pltpu.roll(blk, (Z - s) % Z, axis=1)
        off = pl.multiple_of(r * Z, 128)
        macc_ref[:, pl.ds(off, Z)] = macc_ref[:, pl.ds(off, Z)] + rolled
        return carry

    lax.fori_loop(0, na, body1, 0)

    m = jnp.mod(macc_ref[...], 2.0)
    m0 = m[:, 0 * Z:1 * Z]
    m1 = m[:, 1 * Z:2 * Z]
    m2 = m[:, 2 * Z:3 * Z]
    m3 = m[:, 3 * Z:4 * Z]
    mtot = jnp.mod(m0 + m1 + m2 + m3, 2.0)
    p0 = pltpu.roll(mtot, 1, axis=1)
    p1 = jnp.mod(m1 + m2 + m3, 2.0)
    p3 = jnp.mod(m3 + p0, 2.0)
    p2 = jnp.mod(m2 + p3, 2.0)
    cw_ref[:, K + 0 * Z:K + 1 * Z] = p0
    cw_ref[:, K + 1 * Z:K + 2 * Z] = p1
    cw_ref[:, K + 2 * Z:K + 3 * Z] = p2
    cw_ref[:, K + 3 * Z:K + 4 * Z] = p3

    out_ref[:, :K - 2 * Z] = bits_ref[:, 2 * Z:]
    out_ref[:, K - 2 * Z:K + 2 * Z] = cw_ref[:, K:K + 4 * Z]

    for r in range(EXT_ROWS):
        acc = jnp.zeros((nb, Z), jnp.float32)
        for e in range(4):
            i = 4 * r + e
            c = cc_ref[i]
            s = cs_ref[i]
            blk = cw_ref[:, pl.ds(pl.multiple_of(c * Z, 128), Z)]
            acc = acc + pltpu.roll(blk, (Z - s) % Z, axis=1)
        out_ref[:, K + 2 * Z + r * Z:K + 2 * Z + (r + 1) * Z] = jnp.mod(acc, 2.0)


def kernel(inputs, A_r, A_c, A_s, C_r, C_c, C_s):
    bits_all = inputs.astype(jnp.float32)
    bits = bits_all[:B_SC].reshape(B_SC * K)
    ar = jnp.asarray(A_r, jnp.int32)
    ac = jnp.asarray(A_c, jnp.int32)
    ash = jnp.asarray(A_s, jnp.int32)
    cc = jnp.asarray(C_c, jnp.int32)
    cs = jnp.asarray(C_s, jnp.int32)
    del C_r
    na = ar.shape[0]

    iota16 = jnp.arange(16, dtype=jnp.int32)
    perm = jnp.argsort(ar, stable=True)
    r_sorted = ar[perm]
    first = jnp.searchsorted(r_sorted, jnp.arange(4, dtype=jnp.int32))
    rank = jnp.arange(na, dtype=jnp.int32) - first[r_sorted]
    slots = r_sorted * 22 + rank
    a_base = (ac * DBL + ash)[perm][:, None] + iota16[None, :]
    ab = jnp.full((4 * 22, 16), ZPAD, jnp.int32).at[slots].set(a_base)
    ab = ab.reshape(-1)
    cb = ((cc[:4 * EXT_ROWS] * DBL + cs[:4 * EXT_ROWS])[:, None]
          + iota16[None, :]).reshape(-1)
    iota = jnp.arange(Z, dtype=jnp.int32)
    p0i = (iota + Z - 1) % Z

    mesh = plsc.VectorSubcoreMesh(core_axis_name="c", subcore_axis_name="s")
    out_sc = pl.kernel(
        _sc_body,
        out_type=jax.ShapeDtypeStruct((B_SC * N,), jnp.float32),
        mesh=mesh,
        compiler_params=pltpu.CompilerParams(needs_layout_passes=False),
        scratch_types=[
            pltpu.VMEM((CWD,), jnp.float32),
            pltpu.VMEM((CWD,), jnp.float32),
            pltpu.VMEM((K,), jnp.float32),
            pltpu.VMEM((K,), jnp.float32),
            pltpu.VMEM((4 * 22 * 16,), jnp.int32),
            pltpu.VMEM((4 * EXT_ROWS * 16,), jnp.int32),
            pltpu.VMEM((Z,), jnp.int32),
            pltpu.VMEM((4 * Z,), jnp.float32),
            pltpu.VMEM((Z,), jnp.float32),
            pltpu.VMEM((EXT_ROWS * Z,), jnp.float32),
            pltpu.VMEM((EXT_ROWS * Z,), jnp.float32),
            pltpu.SemaphoreType.DMA,
            pltpu.SemaphoreType.DMA,
            pltpu.SemaphoreType.DMA,
            pltpu.SemaphoreType.DMA,
            pltpu.SemaphoreType.DMA,
        ],
    )(bits, ab, cb, p0i)

    import functools
    body = functools.partial(_tc_body, B_TC, na)
    smem = pl.BlockSpec(memory_space=pltpu.SMEM)
    out_tc = pl.pallas_call(
        body,
        out_shape=jax.ShapeDtypeStruct((B_TC, N), jnp.float32),
        in_specs=[pl.BlockSpec(memory_space=pltpu.VMEM),
                  smem, smem, smem, smem, smem],
        out_specs=pl.BlockSpec(memory_space=pltpu.VMEM),
        scratch_shapes=[pltpu.VMEM((B_TC, 4 * Z), jnp.float32),
                        pltpu.VMEM((B_TC, NBLK * Z), jnp.float32)],
    )(bits_all[B_SC:], ar, ac, ash, cc, cs)

    return jnp.concatenate([out_sc.reshape(B_SC, N), out_tc], axis=0)


# speedup vs baseline: 2.6635x; 1.1350x over previous
import jax
import jax.numpy as jnp
from jax import lax
from jax.experimental import pallas as pl
from jax.experimental.pallas import tpu as pltpu
from jax.experimental.pallas import tpu_sc as plsc

Z = 384
B = 64
K = 8448
N = 16896
EXT_ROWS = 20
NBLK = 26
DBL = 2 * Z
KD = 22 * DBL
ZPAD = NBLK * DBL
CWD = ZPAD + Z
NCHUNK = Z // 16
HALF = NCHUNK // 2

NC = 2
NS = 16
B_SC = 32
B_TC = B - B_SC
ROWS_PER_W = B_SC // (NC * NS)


def _par2(x):
    return (x.astype(jnp.int32) & 1).astype(jnp.float32)


def _sc_body(bits_hbm, ab_hbm, cb_hbm, p0i_hbm, out_hbm,
             cwd0, cwd1, bv0, bv1, ab_v, cb_v, p0i_v, m_v, mt_v, ext0, ext1,
             s_tab, s_in0, s_in1, s_out0, s_out1):
    wid = lax.axis_index("s") * NC + lax.axis_index("c")
    cwds = (cwd0, cwd1)
    bvs = (bv0, bv1)
    exts = (ext0, ext1)
    s_ins = (s_in0, s_in1)
    s_outs = (s_out0, s_out1)

    d_tab = [pltpu.async_copy(ab_hbm, ab_v, s_tab),
             pltpu.async_copy(cb_hbm, cb_v, s_tab),
             pltpu.async_copy(p0i_hbm, p0i_v, s_tab)]
    d_in = []
    for k in range(ROWS_PER_W):
        b = wid * ROWS_PER_W + k
        d_in.append(pltpu.async_copy(
            bits_hbm.at[pl.ds(b * K, K)], bvs[k], s_ins[k]))

    def zero_tail(j, carry):
        cwd0[pl.ds(ZPAD + j * 16, 16)] = jnp.zeros((16,), jnp.float32)
        cwd1[pl.ds(ZPAD + j * 16, 16)] = jnp.zeros((16,), jnp.float32)
        return carry
    lax.fori_loop(0, NCHUNK, zero_tail, 0)
    for d in d_tab:
        d.wait()

    zero16 = jnp.zeros((16,), jnp.float32)
    d_out = []
    for k in range(ROWS_PER_W):
        b = wid * ROWS_PER_W + k
        cwd_v = cwds[k]
        bits_v = bvs[k]
        ext_v = exts[k]
        d_in[k].wait()

        def dup(c, carry, cwd_v=cwd_v, bits_v=bits_v):
            vs = [bits_v[pl.ds(c * Z + j * 16, 16)] for j in range(NCHUNK)]
            for j in range(NCHUNK):
                cwd_v[pl.ds(c * DBL + j * 16, 16)] = vs[j]
                cwd_v[pl.ds(c * DBL + Z + j * 16, 16)] = vs[j]
            return carry
        lax.fori_loop(0, 22, dup, 0)

        for r in range(4):
            for h in range(2):
                def entry1(e, accs, r=r, h=h, cwd_v=cwd_v):
                    base = ab_v[pl.ds((r * 22 + e) * 16, 16)]
                    return tuple(
                        a + plsc.load_gather(cwd_v, [base + (h * HALF + j) * 16])
                        for j, a in enumerate(accs))
                accs = lax.fori_loop(0, 22, entry1, (zero16,) * HALF)
                for j, a in enumerate(accs):
                    m_v[pl.ds(r * Z + (h * HALF + j) * 16, 16)] = _par2(a)

        def stage2(j, carry):
            off = j * 16
            m0 = m_v[pl.ds(0 * Z + off, 16)]
            m1 = m_v[pl.ds(1 * Z + off, 16)]
            m2 = m_v[pl.ds(2 * Z + off, 16)]
            m3 = m_v[pl.ds(3 * Z + off, 16)]
            mt_v[pl.ds(off, 16)] = _par2(m0 + m1 + m2 + m3)
            return carry
        lax.fori_loop(0, NCHUNK, stage2, 0)

        def stage2b(j, carry, cwd_v=cwd_v):
            off = j * 16
            pidx = p0i_v[pl.ds(off, 16)]
            p0 = plsc.load_gather(mt_v, [pidx])
            m1 = m_v[pl.ds(1 * Z + off, 16)]
            m2 = m_v[pl.ds(2 * Z + off, 16)]
            m3 = m_v[pl.ds(3 * Z + off, 16)]
            p1 = _par2(m1 + m2 + m3)
            p3 = _par2(m3 + p0)
            p2 = _par2(m2 + p3)
            for i, p in enumerate((p0, p1, p2, p3)):
                cwd_v[pl.ds((22 + i) * DBL + off, 16)] = p
                cwd_v[pl.ds((22 + i) * DBL + Z + off, 16)] = p
            return carry
        lax.fori_loop(0, NCHUNK, stage2b, 0)

        def ext_row(r, carry, cwd_v=cwd_v, ext_v=ext_v):
            for h in range(2):
                def entry3(e, accs, h=h):
                    base = cb_v[pl.ds((r * 4 + e) * 16, 16)]
                    return tuple(
                        a + plsc.load_gather(cwd_v, [base + (h * HALF + j) * 16])
                        for j, a in enumerate(accs))
                accs = lax.fori_loop(0, 4, entry3, (zero16,) * HALF)
                for j, a in enumerate(accs):
                    ext_v[pl.ds(r * Z + (h * HALF + j) * 16, 16)] = _par2(a)
            return carry
        lax.fori_loop(0, EXT_ROWS, ext_row, 0)

        ob = b * N
        d_out.append(pltpu.async_copy(
            bits_v.at[pl.ds(2 * Z, K - 2 * Z)],
            out_hbm.at[pl.ds(ob, K - 2 * Z)], s_outs[k]))
        for i in range(4):
            d_out.append(pltpu.async_copy(
                cwd_v.at[pl.ds((22 + i) * DBL, Z)],
                out_hbm.at[pl.ds(ob + K - 2 * Z + i * Z, Z)], s_outs[k]))
        d_out.append(pltpu.async_copy(
            ext_v, out_hbm.at[pl.ds(ob + K + 2 * Z, EXT_ROWS * Z)], s_outs[k]))
    for d in d_out:
        d.wait()


def _tc_body(nb, na, bits_ref, ar_ref, ac_ref, as_ref, cc_ref, cs_ref,
             out_ref, macc_ref, cw_ref):
    cw_ref[:, :K] = bits_ref[...]
    macc_ref[...] = jnp.zeros((nb, 4 * Z), jnp.float32)

    def body1(i, carry):
        c = ac_ref[i]
        s = as_ref[i]
        r = ar_ref[i]
        blk = bits_ref[:, pl.ds(pl.multiple_of(c * Z, 128), Z)]
        rolled = pltpu.roll(blk, (Z - s) % Z, axis=1)
        off = pl.multiple_of(r * Z, 128)
        macc_ref[:, pl.ds(off, Z)] = macc_ref[:, pl.ds(off, Z)] + rolled
        return carry

    lax.fori_loop(0, na, body1, 0)

    m = jnp.mod(macc_ref[...], 2.0)
    m0 = m[:, 0 * Z:1 * Z]
    m1 = m[:, 1 * Z:2 * Z]
    m2 = m[:, 2 * Z:3 * Z]
    m3 = m[:, 3 * Z:4 * Z]
    mtot = jnp.mod(m0 + m1 + m2 + m3, 2.0)
    p0 = pltpu.roll(mtot, 1, axis=1)
    p1 = jnp.mod(m1 + m2 + m3, 2.0)
    p3 = jnp.mod(m3 + p0, 2.0)
    p2 = jnp.mod(m2 + p3, 2.0)
    cw_ref[:, K + 0 * Z:K + 1 * Z] = p0
    cw_ref[:, K + 1 * Z:K + 2 * Z] = p1
    cw_ref[:, K + 2 * Z:K + 3 * Z] = p2
    cw_ref[:, K + 3 * Z:K + 4 * Z] = p3

    out_ref[:, :K - 2 * Z] = bits_ref[:, 2 * Z:]
    out_ref[:, K - 2 * Z:K + 2 * Z] = cw_ref[:, K:K + 4 * Z]

    for r in range(EXT_ROWS):
        acc = jnp.zeros((nb, Z), jnp.float32)
        for e in range(4):
            i = 4 * r + e
            c = cc_ref[i]
            s = cs_ref[i]
            blk = cw_ref[:, pl.ds(pl.multiple_of(c * Z, 128), Z)]
            acc = acc + pltpu.roll(blk, (Z - s) % Z, axis=1)
        out_ref[:, K + 2 * Z + r * Z:K + 2 * Z + (r + 1) * Z] = jnp.mod(acc, 2.0)


def kernel(inputs, A_r, A_c, A_s, C_r, C_c, C_s):
    bits_all = inputs.astype(jnp.float32)
    bits = bits_all[:B_SC].reshape(B_SC * K)
    ar = jnp.asarray(A_r, jnp.int32)
    ac = jnp.asarray(A_c, jnp.int32)
    ash = jnp.asarray(A_s, jnp.int32)
    cc = jnp.asarray(C_c, jnp.int32)
    cs = jnp.asarray(C_s, jnp.int32)
    del C_r
    na = ar.shape[0]

    iota16 = jnp.arange(16, dtype=jnp.int32)
    first = jnp.sum(ar[None, :] < jnp.arange(4, dtype=jnp.int32)[:, None],
                    axis=1, dtype=jnp.int32)
    rank = jnp.arange(na, dtype=jnp.int32) - first[ar]
    slots = ar * 22 + rank
    a_base = (ac * DBL + ash)[:, None] + iota16[None, :]
    ab = jnp.full((4 * 22, 16), ZPAD, jnp.int32).at[slots].set(a_base)
    ab = ab.reshape(-1)
    cb = ((cc[:4 * EXT_ROWS] * DBL + cs[:4 * EXT_ROWS])[:, None]
          + iota16[None, :]).reshape(-1)
    iota = jnp.arange(Z, dtype=jnp.int32)
    p0i = (iota + Z - 1) % Z

    mesh = plsc.VectorSubcoreMesh(core_axis_name="c", subcore_axis_name="s")
    out_sc = pl.kernel(
        _sc_body,
        out_type=jax.ShapeDtypeStruct((B_SC * N,), jnp.float32),
        mesh=mesh,
        compiler_params=pltpu.CompilerParams(needs_layout_passes=False),
        scratch_types=[
            pltpu.VMEM((CWD,), jnp.float32),
            pltpu.VMEM((CWD,), jnp.float32),
            pltpu.VMEM((K,), jnp.float32),
            pltpu.VMEM((K,), jnp.float32),
            pltpu.VMEM((4 * 22 * 16,), jnp.int32),
            pltpu.VMEM((4 * EXT_ROWS * 16,), jnp.int32),
            pltpu.VMEM((Z,), jnp.int32),
            pltpu.VMEM((4 * Z,), jnp.float32),
            pltpu.VMEM((Z,), jnp.float32),
            pltpu.VMEM((EXT_ROWS * Z,), jnp.float32),
            pltpu.VMEM((EXT_ROWS * Z,), jnp.float32),
            pltpu.SemaphoreType.DMA,
            pltpu.SemaphoreType.DMA,
            pltpu.SemaphoreType.DMA,
            pltpu.SemaphoreType.DMA,
            pltpu.SemaphoreType.DMA,
        ],
    )(bits, ab, cb, p0i)

    import functools
    body = functools.partial(_tc_body, B_TC, na)
    smem = pl.BlockSpec(memory_space=pltpu.SMEM)
    out_tc = pl.pallas_call(
        body,
        out_shape=jax.ShapeDtypeStruct((B_TC, N), jnp.float32),
        in_specs=[pl.BlockSpec(memory_space=pltpu.VMEM),
                  smem, smem, smem, smem, smem],
        out_specs=pl.BlockSpec(memory_space=pltpu.VMEM),
        scratch_shapes=[pltpu.VMEM((B_TC, 4 * Z), jnp.float32),
                        pltpu.VMEM((B_TC, NBLK * Z), jnp.float32)],
    )(bits_all[B_SC:], ar, ac, ash, cc, cs)

    return jnp.concatenate([out_sc.reshape(B_SC, N), out_tc], axis=0)
